# baseline (speedup 1.0000x reference)
import functools

import jax
import jax.numpy as jnp
from jax import lax
from jax.experimental import pallas as pl
from jax.experimental.pallas import tpu as pltpu
from jax.experimental.pallas import tpu_sc as plsc

N = 10000
E = 320000
D = 128

NC = 2
NS = 16
NW = NC * NS
EPT = E // NW
WIN = 100
NWIN = EPT // WIN
H = D // 2
EPT2 = E // NS
NWIN2 = EPT2 // WIN

_mesh = plsc.VectorSubcoreMesh(core_axis_name="c", subcore_axis_name="s")
_untiled = pltpu.CompilerParams(use_tc_tiling_on_sc=False)


def _sc_degree(dst3, zeros_n16, ones_w16):
  @functools.partial(
      pl.kernel,
      mesh=_mesh,
      compiler_params=_untiled,
      out_type=jax.ShapeDtypeStruct((NC, N, 16), jnp.float32),
      scratch_types=[
          pltpu.VMEM((NWIN, WIN), jnp.int32),
          pltpu.VMEM((WIN, 16), jnp.float32),
          pltpu.VMEM_SHARED((N, 16), jnp.float32),
      ],
  )
  def k(dst_hbm, zeros_hbm, ones_hbm, out_hbm, dst_v, ones_v, acc):
    cid = lax.axis_index("c")
    sid = lax.axis_index("s")
    wid = sid * NC + cid

    @pl.when(sid == 0)
    def _():
      pltpu.sync_copy(zeros_hbm, acc)

    pltpu.sync_copy(dst_hbm.at[wid], dst_v)
    pltpu.sync_copy(ones_hbm, ones_v)
    plsc.subcore_barrier()

    @pl.loop(0, NWIN)
    def _(w):
      pltpu.sync_copy(ones_v, acc.at[dst_v.at[w]], add=True)

    plsc.subcore_barrier()

    @pl.when(sid == 0)
    def _():
      pltpu.sync_copy(acc, out_hbm.at[cid])

  return k(dst3, zeros_n16, ones_w16)


def _sc_scatter(hsl, hsr, src2, dst2, zeros_nh):
  @functools.partial(
      pl.kernel,
      mesh=_mesh,
      compiler_params=_untiled,
      out_type=jax.ShapeDtypeStruct((NC, N, H), jnp.float32),
      scratch_types=[
          pltpu.VMEM((NWIN2, WIN), jnp.int32),
          pltpu.VMEM((NWIN2, WIN), jnp.int32),
          pltpu.VMEM((WIN, H), jnp.float32),
          pltpu.VMEM((WIN, H), jnp.float32),
          pltpu.VMEM_SHARED((N, H), jnp.float32),
          pltpu.SemaphoreType.DMA,
          pltpu.SemaphoreType.DMA,
      ],
  )
  def k(hsl_hbm, hsr_hbm, src_hbm, dst_hbm, zeros_hbm, out_hbm,
        src_v, dst_v, rows0, rows1, acc, sem0, sem1):
    cid = lax.axis_index("c")
    sid = lax.axis_index("s")

    @pl.when(sid == 0)
    def _():
      pltpu.sync_copy(zeros_hbm, acc)

    pltpu.sync_copy(src_hbm.at[sid], src_v)
    pltpu.sync_copy(dst_hbm.at[sid], dst_v)
    plsc.subcore_barrier()

    def run(table):
      pltpu.async_copy(table.at[src_v.at[0]], rows0, sem0)

      @pl.loop(0, NWIN2, step=2)
      def _(w):
        pltpu.async_copy(table.at[src_v.at[w + 1]], rows1, sem1)
        pltpu.make_async_copy(table.at[src_v.at[w]], rows0, sem0).wait()
        pltpu.sync_copy(rows0, acc.at[dst_v.at[w]], add=True)

        @pl.when(w + 2 < NWIN2)
        def _():
          pltpu.async_copy(table.at[src_v.at[w + 2]], rows0, sem0)

        pltpu.make_async_copy(table.at[src_v.at[w + 1]], rows1, sem1).wait()
        pltpu.sync_copy(rows1, acc.at[dst_v.at[w + 1]], add=True)

    @pl.when(cid == 0)
    def _():
      run(hsl_hbm)

    @pl.when(cid == 1)
    def _():
      run(hsr_hbm)

    plsc.subcore_barrier()

    @pl.when(sid == 0)
    def _():
      pltpu.sync_copy(acc, out_hbm.at[cid])

  return k(hsl, hsr, src2, dst2, zeros_nh)


def _sc_gather3(pa, pb, src3, dst3, neg3):
  @functools.partial(
      pl.kernel,
      mesh=_mesh,
      compiler_params=_untiled,
      out_type=(
          jax.ShapeDtypeStruct((E, D), jnp.bfloat16),
          jax.ShapeDtypeStruct((E, D), jnp.bfloat16),
          jax.ShapeDtypeStruct((E, D), jnp.bfloat16),
      ),
      scratch_types=[
          pltpu.VMEM((NWIN, WIN), jnp.int32),
          pltpu.VMEM((WIN, D), jnp.bfloat16),
          pltpu.VMEM((WIN, D), jnp.bfloat16),
          pltpu.SemaphoreType.DMA,
          pltpu.SemaphoreType.DMA,
      ],
  )
  def k(pa_hbm, pb_hbm, src_hbm, dst_hbm, neg_hbm, ga_hbm, gb_hbm, gn_hbm,
        idx_v, rows0, rows1, sem0, sem1):
    cid = lax.axis_index("c")
    sid = lax.axis_index("s")
    wid = sid * NC + cid
    base = wid * EPT

    for table, idx_hbm, out_hbm in ((pa_hbm, src_hbm, ga_hbm),
                                    (pb_hbm, dst_hbm, gb_hbm),
                                    (pb_hbm, neg_hbm, gn_hbm)):
      pltpu.sync_copy(idx_hbm.at[wid], idx_v)
      pltpu.async_copy(table.at[idx_v.at[0]], rows0, sem0)

      @pl.loop(0, NWIN, step=2)
      def _(w, table=table, out_hbm=out_hbm):
        pltpu.async_copy(table.at[idx_v.at[w + 1]], rows1, sem1)
        pltpu.make_async_copy(table.at[idx_v.at[w]], rows0, sem0).wait()
        pltpu.sync_copy(rows0, out_hbm.at[pl.ds(base + w * WIN, WIN)])

        @pl.when(w + 2 < NWIN)
        def _():
          pltpu.async_copy(table.at[idx_v.at[w + 2]], rows0, sem0)

        pltpu.make_async_copy(table.at[idx_v.at[w + 1]], rows1, sem1).wait()
        pltpu.sync_copy(rows1, out_hbm.at[pl.ds(base + (w + 1) * WIN, WIN)])

  return k(pa, pb, src3, dst3, neg3)


def _tc_prep_body(deg_ref, x_ref, w_ref, hsl_ref, hsr_ref, dinv_ref):
  deg = deg_ref[0] + deg_ref[1]
  total = deg[:, 0:1] + 1.0
  dinv = lax.rsqrt(total)
  h = jnp.dot(x_ref[...], w_ref[...], preferred_element_type=jnp.float32)
  hs = h * dinv
  hsl_ref[...] = hs[:, :H]
  hsr_ref[...] = hs[:, H:]
  dinv_ref[...] = dinv


def _tc_prep(deg_p, x, w1):
  return pl.pallas_call(
      _tc_prep_body,
      out_shape=(jax.ShapeDtypeStruct((N, H), jnp.float32),
                 jax.ShapeDtypeStruct((N, H), jnp.float32),
                 jax.ShapeDtypeStruct((N, 1), jnp.float32)),
  )(deg_p, x, w1)


def _node_state(p_ref, hsl_ref, hsr_ref, dinv, b_ref):
  hs = jnp.concatenate([hsl_ref[...], hsr_ref[...]], axis=1)
  psum = jnp.concatenate([p_ref[0], p_ref[1]], axis=1)
  return dinv * (psum + hs) + b_ref[...]


def _tc_layer_body(p_ref, hsl_ref, hsr_ref, dinv_ref, b_ref, g_ref, be_ref,
                   wn_ref, outl_ref, outr_ref):
  dinv = dinv_ref[...]
  t = _node_state(p_ref, hsl_ref, hsr_ref, dinv, b_ref)
  m = jnp.mean(t, axis=0, keepdims=True)
  c = t - m
  v = jnp.mean(c * c, axis=0, keepdims=True)
  y = c * lax.rsqrt(v + 1e-5) * g_ref[...] + be_ref[...]
  y = jnp.maximum(y, 0.0)
  hs = jnp.dot(y, wn_ref[...], preferred_element_type=jnp.float32) * dinv
  outl_ref[...] = hs[:, :H]
  outr_ref[...] = hs[:, H:]


def _tc_layer(p, hsl, hsr, dinv, b, g, be, wn):
  return pl.pallas_call(
      _tc_layer_body,
      out_shape=(jax.ShapeDtypeStruct((N, H), jnp.float32),
                 jax.ShapeDtypeStruct((N, H), jnp.float32)),
  )(p, hsl, hsr, dinv, b, g, be, wn)


def _tc_final_body(p_ref, hsl_ref, hsr_ref, dinv_ref, b_ref, w1_ref, b1_ref,
                   pa_ref, pb_ref):
  dinv = dinv_ref[...]
  z = _node_state(p_ref, hsl_ref, hsr_ref, dinv, b_ref)
  pa = jnp.dot(z, w1_ref[:D], preferred_element_type=jnp.float32) + b1_ref[...]
  pb = jnp.dot(z, w1_ref[D:], preferred_element_type=jnp.float32)
  pa_ref[...] = pa.astype(jnp.bfloat16)
  pb_ref[...] = pb.astype(jnp.bfloat16)


def _tc_final(p, hsl, hsr, dinv, b3, fc1_w, fc1_b):
  return pl.pallas_call(
      _tc_final_body,
      out_shape=(jax.ShapeDtypeStruct((N, D), jnp.bfloat16),
                 jax.ShapeDtypeStruct((N, D), jnp.bfloat16)),
  )(p, hsl, hsr, dinv, b3, fc1_w, fc1_b)


_BE = 8000


def _tc_decode_body(ga_ref, gb_ref, gn_ref, w_ref, c_ref, pos_ref, neg_ref):
  ga = ga_ref[...].astype(jnp.float32)
  w = w_ref[...]
  c = c_ref[0, 0]
  hp = jnp.maximum(ga + gb_ref[...].astype(jnp.float32), 0.0)
  hn = jnp.maximum(ga + gn_ref[...].astype(jnp.float32), 0.0)
  pos_ref[...] = jax.nn.sigmoid(
      jnp.dot(hp, w, preferred_element_type=jnp.float32) + c)
  neg_ref[...] = jax.nn.sigmoid(
      jnp.dot(hn, w, preferred_element_type=jnp.float32) + c)


def _tc_decode(ga, gb, gn, fc2_w, fc2_b):
  grid = (E // _BE,)
  edge_spec = pl.BlockSpec((_BE, D), lambda i: (i, 0))
  return pl.pallas_call(
      _tc_decode_body,
      grid=grid,
      in_specs=[edge_spec, edge_spec, edge_spec,
                pl.BlockSpec((D, 1), lambda i: (0, 0)),
                pl.BlockSpec((1, 1), lambda i: (0, 0))],
      out_specs=(pl.BlockSpec((_BE, 1), lambda i: (i, 0)),
                 pl.BlockSpec((_BE, 1), lambda i: (i, 0))),
      out_shape=(jax.ShapeDtypeStruct((E, 1), jnp.float32),
                 jax.ShapeDtypeStruct((E, 1), jnp.float32)),
  )(ga, gb, gn, fc2_w, fc2_b)


def kernel(node_feat, src, dst, neg, W1, b1, W2, b2, W3, b3,
           g1, be1, g2, be2, fc1_W, fc1_b, fc2_W, fc2_b):
  src3 = src.reshape(NW, NWIN, WIN)
  dst3 = dst.reshape(NW, NWIN, WIN)
  neg3 = neg.reshape(NW, NWIN, WIN)
  src2 = src.reshape(NS, NWIN2, WIN)
  dst2 = dst.reshape(NS, NWIN2, WIN)

  zeros_nh = jnp.zeros((N, H), jnp.float32)
  zeros_n16 = jnp.zeros((N, 16), jnp.float32)
  ones_w16 = jnp.ones((WIN, 16), jnp.float32)

  deg_p = _sc_degree(dst3, zeros_n16, ones_w16)
  hs1l, hs1r, dinv = _tc_prep(deg_p, node_feat, W1)
  p1 = _sc_scatter(hs1l, hs1r, src2, dst2, zeros_nh)
  hs2l, hs2r = _tc_layer(p1, hs1l, hs1r, dinv, b1, g1, be1, W2)
  p2 = _sc_scatter(hs2l, hs2r, src2, dst2, zeros_nh)
  hs3l, hs3r = _tc_layer(p2, hs2l, hs2r, dinv, b2, g2, be2, W3)
  p3 = _sc_scatter(hs3l, hs3r, src2, dst2, zeros_nh)
  pa, pb = _tc_final(p3, hs3l, hs3r, dinv, b3, fc1_W, fc1_b)
  ga, gb, gn = _sc_gather3(pa, pb, src3, dst3, neg3)
  pos, negv = _tc_decode(ga, gb, gn, fc2_W, fc2_b.reshape(1, 1))
  return pos.reshape(-1), negv.reshape(-1)


# speedup vs baseline: 1.0592x; 1.0592x over previous
import functools

import jax
import jax.numpy as jnp
from jax import lax
from jax.experimental import pallas as pl
from jax.experimental.pallas import tpu as pltpu
from jax.experimental.pallas import tpu_sc as plsc

N = 10000
E = 320000
D = 128

NC = 2
NS = 16
NW = NC * NS
EPT = E // NW
WIN = 100
NWIN = EPT // WIN
H = D // 2
EPT2 = E // NS
NWIN2 = EPT2 // WIN

_mesh = plsc.VectorSubcoreMesh(core_axis_name="c", subcore_axis_name="s")
_untiled = pltpu.CompilerParams(use_tc_tiling_on_sc=False)


def _sc_degree(dst3, zeros_n16, ones_w16):
  @functools.partial(
      pl.kernel,
      mesh=_mesh,
      compiler_params=_untiled,
      out_type=jax.ShapeDtypeStruct((NC, N, 16), jnp.float32),
      scratch_types=[
          pltpu.VMEM((NWIN, WIN), jnp.int32),
          pltpu.VMEM((WIN, 16), jnp.float32),
          pltpu.VMEM_SHARED((N, 16), jnp.float32),
          pltpu.SemaphoreType.DMA,
      ],
  )
  def k(dst_hbm, zeros_hbm, ones_hbm, out_hbm, dst_v, ones_v, acc, sem):
    cid = lax.axis_index("c")
    sid = lax.axis_index("s")
    wid = sid * NC + cid

    @pl.when(sid == 0)
    def _():
      pltpu.sync_copy(zeros_hbm, acc)

    pltpu.sync_copy(dst_hbm.at[wid], dst_v)
    pltpu.sync_copy(ones_hbm, ones_v)
    plsc.subcore_barrier()

    @pl.loop(0, NWIN)
    def _(w):
      pltpu.async_copy(ones_v, acc.at[dst_v.at[w]], sem, add=True)

    @pl.loop(0, NWIN)
    def _(w):
      pltpu.make_async_copy(ones_v, acc.at[dst_v.at[0]], sem).wait()

    plsc.subcore_barrier()

    @pl.when(sid == 0)
    def _():
      pltpu.sync_copy(acc, out_hbm.at[cid])

  return k(dst3, zeros_n16, ones_w16)


def _sc_scatter(hsl, hsr, src2, dst2, zeros_nh):
  @functools.partial(
      pl.kernel,
      mesh=_mesh,
      compiler_params=_untiled,
      out_type=jax.ShapeDtypeStruct((NC, N, H), jnp.float32),
      scratch_types=[
          pltpu.VMEM((NWIN2, WIN), jnp.int32),
          pltpu.VMEM((NWIN2, WIN), jnp.int32),
          pltpu.VMEM((WIN, H), jnp.float32),
          pltpu.VMEM((WIN, H), jnp.float32),
          pltpu.VMEM((WIN, H), jnp.float32),
          pltpu.VMEM((WIN, H), jnp.float32),
          pltpu.VMEM_SHARED((N, H), jnp.float32),
          pltpu.SemaphoreType.DMA,
          pltpu.SemaphoreType.DMA,
          pltpu.SemaphoreType.DMA,
          pltpu.SemaphoreType.DMA,
          pltpu.SemaphoreType.DMA,
          pltpu.SemaphoreType.DMA,
          pltpu.SemaphoreType.DMA,
          pltpu.SemaphoreType.DMA,
      ],
  )
  def k(hsl_hbm, hsr_hbm, src_hbm, dst_hbm, zeros_hbm, out_hbm,
        src_v, dst_v, r0, r1, r2, r3, acc,
        g0, g1, g2, g3, s0, s1, s2, s3):
    cid = lax.axis_index("c")
    sid = lax.axis_index("s")
    rows = (r0, r1, r2, r3)
    gsem = (g0, g1, g2, g3)
    ssem = (s0, s1, s2, s3)

    @pl.when(sid == 0)
    def _():
      pltpu.sync_copy(zeros_hbm, acc)

    pltpu.sync_copy(src_hbm.at[sid], src_v)
    pltpu.sync_copy(dst_hbm.at[sid], dst_v)
    plsc.subcore_barrier()

    def run(table):
      for b in range(4):
        pltpu.async_copy(table.at[src_v.at[b]], rows[b], gsem[b])

      @pl.loop(0, NWIN2, step=4)
      def _(w):
        for b in range(4):
          pltpu.make_async_copy(
              table.at[src_v.at[w + b]], rows[b], gsem[b]).wait()
          pltpu.async_copy(
              rows[b], acc.at[dst_v.at[w + b]], ssem[b], add=True)
        for b in range(4):
          pltpu.make_async_copy(
              rows[b], acc.at[dst_v.at[w + b]], ssem[b]).wait()

          @pl.when(w + b + 4 < NWIN2)
          def _(b=b):
            pltpu.async_copy(
                table.at[src_v.at[w + b + 4]], rows[b], gsem[b])

    @pl.when(cid == 0)
    def _():
      run(hsl_hbm)

    @pl.when(cid == 1)
    def _():
      run(hsr_hbm)

    plsc.subcore_barrier()

    @pl.when(sid == 0)
    def _():
      pltpu.sync_copy(acc, out_hbm.at[cid])

  return k(hsl, hsr, src2, dst2, zeros_nh)


def _sc_gather3(pa, pb, src3, dst3, neg3):
  @functools.partial(
      pl.kernel,
      mesh=_mesh,
      compiler_params=_untiled,
      out_type=(
          jax.ShapeDtypeStruct((E, D), jnp.bfloat16),
          jax.ShapeDtypeStruct((E, D), jnp.bfloat16),
          jax.ShapeDtypeStruct((E, D), jnp.bfloat16),
      ),
      scratch_types=[
          pltpu.VMEM((NWIN, WIN), jnp.int32),
          pltpu.VMEM((WIN, D), jnp.bfloat16),
          pltpu.VMEM((WIN, D), jnp.bfloat16),
          pltpu.VMEM((WIN, D), jnp.bfloat16),
          pltpu.VMEM((WIN, D), jnp.bfloat16),
          pltpu.SemaphoreType.DMA,
          pltpu.SemaphoreType.DMA,
          pltpu.SemaphoreType.DMA,
          pltpu.SemaphoreType.DMA,
          pltpu.SemaphoreType.DMA,
          pltpu.SemaphoreType.DMA,
          pltpu.SemaphoreType.DMA,
          pltpu.SemaphoreType.DMA,
      ],
  )
  def k(pa_hbm, pb_hbm, src_hbm, dst_hbm, neg_hbm, ga_hbm, gb_hbm, gn_hbm,
        idx_v, r0, r1, r2, r3, g0, g1, g2, g3, s0, s1, s2, s3):
    cid = lax.axis_index("c")
    sid = lax.axis_index("s")
    wid = sid * NC + cid
    base = wid * EPT
    rows = (r0, r1, r2, r3)
    gsem = (g0, g1, g2, g3)
    ssem = (s0, s1, s2, s3)

    for table, idx_hbm, out_hbm in ((pa_hbm, src_hbm, ga_hbm),
                                    (pb_hbm, dst_hbm, gb_hbm),
                                    (pb_hbm, neg_hbm, gn_hbm)):
      pltpu.sync_copy(idx_hbm.at[wid], idx_v)
      for b in range(4):
        pltpu.async_copy(table.at[idx_v.at[b]], rows[b], gsem[b])

      @pl.loop(0, NWIN, step=4)
      def _(w, table=table, out_hbm=out_hbm):
        for b in range(4):
          pltpu.make_async_copy(
              table.at[idx_v.at[w + b]], rows[b], gsem[b]).wait()
          pltpu.async_copy(
              rows[b], out_hbm.at[pl.ds(base + (w + b) * WIN, WIN)], ssem[b])
        for b in range(4):
          pltpu.make_async_copy(
              rows[b], out_hbm.at[pl.ds(base + (w + b) * WIN, WIN)],
              ssem[b]).wait()

          @pl.when(w + b + 4 < NWIN)
          def _(b=b, table=table):
            pltpu.async_copy(table.at[idx_v.at[w + b + 4]], rows[b], gsem[b])

  return k(pa, pb, src3, dst3, neg3)


def _tc_prep_body(deg_ref, x_ref, w_ref, hsl_ref, hsr_ref, dinv_ref):
  deg = deg_ref[0] + deg_ref[1]
  total = deg[:, 0:1] + 1.0
  dinv = lax.rsqrt(total)
  h = jnp.dot(x_ref[...], w_ref[...], preferred_element_type=jnp.float32)
  hs = h * dinv
  hsl_ref[...] = hs[:, :H]
  hsr_ref[...] = hs[:, H:]
  dinv_ref[...] = dinv


def _tc_prep(deg_p, x, w1):
  return pl.pallas_call(
      _tc_prep_body,
      out_shape=(jax.ShapeDtypeStruct((N, H), jnp.float32),
                 jax.ShapeDtypeStruct((N, H), jnp.float32),
                 jax.ShapeDtypeStruct((N, 1), jnp.float32)),
  )(deg_p, x, w1)


def _node_state(p_ref, hsl_ref, hsr_ref, dinv, b_ref):
  hs = jnp.concatenate([hsl_ref[...], hsr_ref[...]], axis=1)
  psum = jnp.concatenate([p_ref[0], p_ref[1]], axis=1)
  return dinv * (psum + hs) + b_ref[...]


def _tc_layer_body(p_ref, hsl_ref, hsr_ref, dinv_ref, b_ref, g_ref, be_ref,
                   wn_ref, outl_ref, outr_ref):
  dinv = dinv_ref[...]
  t = _node_state(p_ref, hsl_ref, hsr_ref, dinv, b_ref)
  m = jnp.mean(t, axis=0, keepdims=True)
  c = t - m
  v = jnp.mean(c * c, axis=0, keepdims=True)
  y = c * lax.rsqrt(v + 1e-5) * g_ref[...] + be_ref[...]
  y = jnp.maximum(y, 0.0)
  hs = jnp.dot(y, wn_ref[...], preferred_element_type=jnp.float32) * dinv
  outl_ref[...] = hs[:, :H]
  outr_ref[...] = hs[:, H:]


def _tc_layer(p, hsl, hsr, dinv, b, g, be, wn):
  return pl.pallas_call(
      _tc_layer_body,
      out_shape=(jax.ShapeDtypeStruct((N, H), jnp.float32),
                 jax.ShapeDtypeStruct((N, H), jnp.float32)),
  )(p, hsl, hsr, dinv, b, g, be, wn)


def _tc_final_body(p_ref, hsl_ref, hsr_ref, dinv_ref, b_ref, w1_ref, b1_ref,
                   pa_ref, pb_ref):
  dinv = dinv_ref[...]
  z = _node_state(p_ref, hsl_ref, hsr_ref, dinv, b_ref)
  pa = jnp.dot(z, w1_ref[:D], preferred_element_type=jnp.float32) + b1_ref[...]
  pb = jnp.dot(z, w1_ref[D:], preferred_element_type=jnp.float32)
  pa_ref[...] = pa.astype(jnp.bfloat16)
  pb_ref[...] = pb.astype(jnp.bfloat16)


def _tc_final(p, hsl, hsr, dinv, b3, fc1_w, fc1_b):
  return pl.pallas_call(
      _tc_final_body,
      out_shape=(jax.ShapeDtypeStruct((N, D), jnp.bfloat16),
                 jax.ShapeDtypeStruct((N, D), jnp.bfloat16)),
  )(p, hsl, hsr, dinv, b3, fc1_w, fc1_b)


_BE = 8000


def _tc_decode_body(ga_ref, gb_ref, gn_ref, w_ref, c_ref, pos_ref, neg_ref):
  ga = ga_ref[...].astype(jnp.float32)
  w = w_ref[...]
  c = c_ref[0, 0]
  hp = jnp.maximum(ga + gb_ref[...].astype(jnp.float32), 0.0)
  hn = jnp.maximum(ga + gn_ref[...].astype(jnp.float32), 0.0)
  pos_ref[...] = jax.nn.sigmoid(
      jnp.dot(hp, w, preferred_element_type=jnp.float32) + c)
  neg_ref[...] = jax.nn.sigmoid(
      jnp.dot(hn, w, preferred_element_type=jnp.float32) + c)


def _tc_decode(ga, gb, gn, fc2_w, fc2_b):
  grid = (E // _BE,)
  edge_spec = pl.BlockSpec((_BE, D), lambda i: (i, 0))
  return pl.pallas_call(
      _tc_decode_body,
      grid=grid,
      in_specs=[edge_spec, edge_spec, edge_spec,
                pl.BlockSpec((D, 1), lambda i: (0, 0)),
                pl.BlockSpec((1, 1), lambda i: (0, 0))],
      out_specs=(pl.BlockSpec((_BE, 1), lambda i: (i, 0)),
                 pl.BlockSpec((_BE, 1), lambda i: (i, 0))),
      out_shape=(jax.ShapeDtypeStruct((E, 1), jnp.float32),
                 jax.ShapeDtypeStruct((E, 1), jnp.float32)),
  )(ga, gb, gn, fc2_w, fc2_b)


def kernel(node_feat, src, dst, neg, W1, b1, W2, b2, W3, b3,
           g1, be1, g2, be2, fc1_W, fc1_b, fc2_W, fc2_b):
  src3 = src.reshape(NW, NWIN, WIN)
  dst3 = dst.reshape(NW, NWIN, WIN)
  neg3 = neg.reshape(NW, NWIN, WIN)
  src2 = src.reshape(NS, NWIN2, WIN)
  dst2 = dst.reshape(NS, NWIN2, WIN)

  zeros_nh = jnp.zeros((N, H), jnp.float32)
  zeros_n16 = jnp.zeros((N, 16), jnp.float32)
  ones_w16 = jnp.ones((WIN, 16), jnp.float32)

  deg_p = _sc_degree(dst3, zeros_n16, ones_w16)
  hs1l, hs1r, dinv = _tc_prep(deg_p, node_feat, W1)
  p1 = _sc_scatter(hs1l, hs1r, src2, dst2, zeros_nh)
  hs2l, hs2r = _tc_layer(p1, hs1l, hs1r, dinv, b1, g1, be1, W2)
  p2 = _sc_scatter(hs2l, hs2r, src2, dst2, zeros_nh)
  hs3l, hs3r = _tc_layer(p2, hs2l, hs2r, dinv, b2, g2, be2, W3)
  p3 = _sc_scatter(hs3l, hs3r, src2, dst2, zeros_nh)
  pa, pb = _tc_final(p3, hs3l, hs3r, dinv, b3, fc1_W, fc1_b)
  ga, gb, gn = _sc_gather3(pa, pb, src3, dst3, neg3)
  pos, negv = _tc_decode(ga, gb, gn, fc2_W, fc2_b.reshape(1, 1))
  return pos.reshape(-1), negv.reshape(-1)


# speedup vs baseline: 1.7574x; 1.6592x over previous
import functools

import jax
import jax.numpy as jnp
from jax import lax
from jax.experimental import pallas as pl
from jax.experimental.pallas import tpu as pltpu
from jax.experimental.pallas import tpu_sc as plsc

N = 10000
E = 320000
D = 128

NC = 2
NS = 16
NW = NC * NS
EPT = E // NW
WIN = 100
NWIN = EPT // WIN
H = D // 2
EPT2 = E // NS
NWIN2 = EPT2 // WIN

_mesh = plsc.VectorSubcoreMesh(core_axis_name="c", subcore_axis_name="s")
_untiled = pltpu.CompilerParams(use_tc_tiling_on_sc=False)


def _sc_degree(dst3, zeros_n16, ones_w16):
  @functools.partial(
      pl.kernel,
      mesh=_mesh,
      compiler_params=_untiled,
      out_type=jax.ShapeDtypeStruct((NC, N, 16), jnp.float32),
      scratch_types=[
          pltpu.VMEM((NWIN, WIN), jnp.int32),
          pltpu.VMEM((WIN, 16), jnp.float32),
          pltpu.VMEM_SHARED((N, 16), jnp.float32),
          pltpu.SemaphoreType.DMA,
      ],
  )
  def k(dst_hbm, zeros_hbm, ones_hbm, out_hbm, dst_v, ones_v, acc, sem):
    cid = lax.axis_index("c")
    sid = lax.axis_index("s")
    wid = sid * NC + cid

    @pl.when(sid == 0)
    def _():
      pltpu.sync_copy(zeros_hbm, acc)

    pltpu.sync_copy(dst_hbm.at[wid], dst_v)
    pltpu.sync_copy(ones_hbm, ones_v)
    plsc.subcore_barrier()

    @pl.loop(0, NWIN)
    def _(w):
      pltpu.async_copy(ones_v, acc.at[dst_v.at[w]], sem, add=True)

    @pl.loop(0, NWIN)
    def _(w):
      pltpu.make_async_copy(ones_v, acc.at[dst_v.at[0]], sem).wait()

    plsc.subcore_barrier()

    @pl.when(sid == 0)
    def _():
      pltpu.sync_copy(acc, out_hbm.at[cid])

  return k(dst3, zeros_n16, ones_w16)


def _sc_scatter(hsl, hsr, src2, dst2, zeros_nh):
  @functools.partial(
      pl.kernel,
      mesh=_mesh,
      compiler_params=_untiled,
      out_type=jax.ShapeDtypeStruct((NC, N, H), jnp.float32),
      scratch_types=[
          pltpu.VMEM((NWIN2, WIN), jnp.int32),
          pltpu.VMEM((NWIN2, WIN), jnp.int32),
          pltpu.VMEM((WIN, H), jnp.float32),
          pltpu.VMEM((WIN, H), jnp.float32),
          pltpu.VMEM((WIN, H), jnp.float32),
          pltpu.VMEM((WIN, H), jnp.float32),
          pltpu.VMEM_SHARED((N, H), jnp.float32),
          pltpu.SemaphoreType.DMA,
          pltpu.SemaphoreType.DMA,
          pltpu.SemaphoreType.DMA,
          pltpu.SemaphoreType.DMA,
          pltpu.SemaphoreType.DMA,
          pltpu.SemaphoreType.DMA,
          pltpu.SemaphoreType.DMA,
          pltpu.SemaphoreType.DMA,
      ],
  )
  def k(hsl_hbm, hsr_hbm, src_hbm, dst_hbm, zeros_hbm, out_hbm,
        src_v, dst_v, r0, r1, r2, r3, acc,
        g0, g1, g2, g3, s0, s1, s2, s3):
    cid = lax.axis_index("c")
    sid = lax.axis_index("s")
    rows = (r0, r1, r2, r3)
    gsem = (g0, g1, g2, g3)
    ssem = (s0, s1, s2, s3)

    @pl.when(sid == 0)
    def _():
      pltpu.sync_copy(zeros_hbm, acc)

    pltpu.sync_copy(src_hbm.at[sid], src_v)
    pltpu.sync_copy(dst_hbm.at[sid], dst_v)
    plsc.subcore_barrier()

    def run(table):
      for b in range(4):
        pltpu.async_copy(table.at[src_v.at[b]], rows[b], gsem[b])

      @pl.loop(0, NWIN2, step=4)
      def _(w):
        for b in range(4):
          pltpu.make_async_copy(
              table.at[src_v.at[w + b]], rows[b], gsem[b]).wait()
          pltpu.async_copy(
              rows[b], acc.at[dst_v.at[w + b]], ssem[b], add=True)
        for b in range(4):
          pltpu.make_async_copy(
              rows[b], acc.at[dst_v.at[w + b]], ssem[b]).wait()

          @pl.when(w + b + 4 < NWIN2)
          def _(b=b):
            pltpu.async_copy(
                table.at[src_v.at[w + b + 4]], rows[b], gsem[b])

    @pl.when(cid == 0)
    def _():
      run(hsl_hbm)

    @pl.when(cid == 1)
    def _():
      run(hsr_hbm)

    plsc.subcore_barrier()

    @pl.when(sid == 0)
    def _():
      pltpu.sync_copy(acc, out_hbm.at[cid])

  return k(hsl, hsr, src2, dst2, zeros_nh)


def _sc_gather3(pa, pb, src3, dst3, neg3):
  @functools.partial(
      pl.kernel,
      mesh=_mesh,
      compiler_params=_untiled,
      out_type=(
          jax.ShapeDtypeStruct((E, D), jnp.float32),
          jax.ShapeDtypeStruct((E, D), jnp.float32),
          jax.ShapeDtypeStruct((E, D), jnp.float32),
      ),
      scratch_types=[
          pltpu.VMEM((NWIN, WIN), jnp.int32),
          pltpu.VMEM((WIN, D), jnp.float32),
          pltpu.VMEM((WIN, D), jnp.float32),
          pltpu.VMEM((WIN, D), jnp.float32),
          pltpu.VMEM((WIN, D), jnp.float32),
          pltpu.SemaphoreType.DMA,
          pltpu.SemaphoreType.DMA,
          pltpu.SemaphoreType.DMA,
          pltpu.SemaphoreType.DMA,
          pltpu.SemaphoreType.DMA,
          pltpu.SemaphoreType.DMA,
          pltpu.SemaphoreType.DMA,
          pltpu.SemaphoreType.DMA,
      ],
  )
  def k(pa_hbm, pb_hbm, src_hbm, dst_hbm, neg_hbm, ga_hbm, gb_hbm, gn_hbm,
        idx_v, r0, r1, r2, r3, g0, g1, g2, g3, s0, s1, s2, s3):
    cid = lax.axis_index("c")
    sid = lax.axis_index("s")
    wid = sid * NC + cid
    base = wid * EPT
    rows = (r0, r1, r2, r3)
    gsem = (g0, g1, g2, g3)
    ssem = (s0, s1, s2, s3)

    for table, idx_hbm, out_hbm in ((pa_hbm, src_hbm, ga_hbm),
                                    (pb_hbm, dst_hbm, gb_hbm),
                                    (pb_hbm, neg_hbm, gn_hbm)):
      pltpu.sync_copy(idx_hbm.at[wid], idx_v)
      for b in range(4):
        pltpu.async_copy(table.at[idx_v.at[b]], rows[b], gsem[b])

      @pl.loop(0, NWIN, step=4)
      def _(w, table=table, out_hbm=out_hbm):
        for b in range(4):
          pltpu.make_async_copy(
              table.at[idx_v.at[w + b]], rows[b], gsem[b]).wait()
          pltpu.async_copy(
              rows[b], out_hbm.at[pl.ds(base + (w + b) * WIN, WIN)], ssem[b])
        for b in range(4):
          pltpu.make_async_copy(
              rows[b], out_hbm.at[pl.ds(base + (w + b) * WIN, WIN)],
              ssem[b]).wait()

          @pl.when(w + b + 4 < NWIN)
          def _(b=b, table=table):
            pltpu.async_copy(table.at[idx_v.at[w + b + 4]], rows[b], gsem[b])

  return k(pa, pb, src3, dst3, neg3)


def _tc_prep_body(deg_ref, x_ref, w_ref, hsl_ref, hsr_ref, dinv_ref):
  deg = deg_ref[0] + deg_ref[1]
  total = deg[:, 0:1] + 1.0
  dinv = lax.rsqrt(total)
  h = jnp.dot(x_ref[...], w_ref[...], preferred_element_type=jnp.float32)
  hs = h * dinv
  hsl_ref[...] = hs[:, :H]
  hsr_ref[...] = hs[:, H:]
  dinv_ref[...] = dinv


def _tc_prep(deg_p, x, w1):
  return pl.pallas_call(
      _tc_prep_body,
      out_shape=(jax.ShapeDtypeStruct((N, H), jnp.float32),
                 jax.ShapeDtypeStruct((N, H), jnp.float32),
                 jax.ShapeDtypeStruct((N, 1), jnp.float32)),
  )(deg_p, x, w1)


def _node_state(p_ref, hsl_ref, hsr_ref, dinv, b_ref):
  hs = jnp.concatenate([hsl_ref[...], hsr_ref[...]], axis=1)
  psum = jnp.concatenate([p_ref[0], p_ref[1]], axis=1)
  return dinv * (psum + hs) + b_ref[...]


def _tc_layer_body(p_ref, hsl_ref, hsr_ref, dinv_ref, b_ref, g_ref, be_ref,
                   wn_ref, outl_ref, outr_ref):
  dinv = dinv_ref[...]
  t = _node_state(p_ref, hsl_ref, hsr_ref, dinv, b_ref)
  m = jnp.mean(t, axis=0, keepdims=True)
  c = t - m
  v = jnp.mean(c * c, axis=0, keepdims=True)
  y = c * lax.rsqrt(v + 1e-5) * g_ref[...] + be_ref[...]
  y = jnp.maximum(y, 0.0)
  hs = jnp.dot(y, wn_ref[...], preferred_element_type=jnp.float32) * dinv
  outl_ref[...] = hs[:, :H]
  outr_ref[...] = hs[:, H:]


def _tc_layer(p, hsl, hsr, dinv, b, g, be, wn):
  return pl.pallas_call(
      _tc_layer_body,
      out_shape=(jax.ShapeDtypeStruct((N, H), jnp.float32),
                 jax.ShapeDtypeStruct((N, H), jnp.float32)),
  )(p, hsl, hsr, dinv, b, g, be, wn)


def _tc_final_body(p_ref, hsl_ref, hsr_ref, dinv_ref, b_ref, w1_ref, b1_ref,
                   pa_ref, pb_ref):
  dinv = dinv_ref[...]
  z = _node_state(p_ref, hsl_ref, hsr_ref, dinv, b_ref)
  pa_ref[...] = jnp.dot(z, w1_ref[:D],
                        preferred_element_type=jnp.float32) + b1_ref[...]
  pb_ref[...] = jnp.dot(z, w1_ref[D:], preferred_element_type=jnp.float32)


def _tc_final(p, hsl, hsr, dinv, b3, fc1_w, fc1_b):
  return pl.pallas_call(
      _tc_final_body,
      out_shape=(jax.ShapeDtypeStruct((N, D), jnp.float32),
                 jax.ShapeDtypeStruct((N, D), jnp.float32)),
  )(p, hsl, hsr, dinv, b3, fc1_w, fc1_b)


_BE = 8000


def _tc_decode_body(ga_ref, gb_ref, gn_ref, w_ref, c_ref, pos_ref, neg_ref):
  ga = ga_ref[...]
  w = w_ref[...]
  c = c_ref[0, 0]
  hp = jnp.maximum(ga + gb_ref[...], 0.0)
  hn = jnp.maximum(ga + gn_ref[...], 0.0)
  pos_ref[...] = jax.nn.sigmoid(
      jnp.dot(hp, w, preferred_element_type=jnp.float32) + c)
  neg_ref[...] = jax.nn.sigmoid(
      jnp.dot(hn, w, preferred_element_type=jnp.float32) + c)


def _tc_decode(ga, gb, gn, fc2_w, fc2_b):
  grid = (E // _BE,)
  edge_spec = pl.BlockSpec((_BE, D), lambda i: (i, 0))
  return pl.pallas_call(
      _tc_decode_body,
      grid=grid,
      in_specs=[edge_spec, edge_spec, edge_spec,
                pl.BlockSpec((D, 1), lambda i: (0, 0)),
                pl.BlockSpec((1, 1), lambda i: (0, 0))],
      out_specs=(pl.BlockSpec((_BE, 1), lambda i: (i, 0)),
                 pl.BlockSpec((_BE, 1), lambda i: (i, 0))),
      out_shape=(jax.ShapeDtypeStruct((E, 1), jnp.float32),
                 jax.ShapeDtypeStruct((E, 1), jnp.float32)),
  )(ga, gb, gn, fc2_w, fc2_b)


def kernel(node_feat, src, dst, neg, W1, b1, W2, b2, W3, b3,
           g1, be1, g2, be2, fc1_W, fc1_b, fc2_W, fc2_b):
  src3 = src.reshape(NW, NWIN, WIN)
  dst3 = dst.reshape(NW, NWIN, WIN)
  neg3 = neg.reshape(NW, NWIN, WIN)
  src2 = src.reshape(NS, NWIN2, WIN)
  dst2 = dst.reshape(NS, NWIN2, WIN)

  zeros_nh = jnp.zeros((N, H), jnp.float32)
  zeros_n16 = jnp.zeros((N, 16), jnp.float32)
  ones_w16 = jnp.ones((WIN, 16), jnp.float32)

  deg_p = _sc_degree(dst3, zeros_n16, ones_w16)
  hs1l, hs1r, dinv = _tc_prep(deg_p, node_feat, W1)
  p1 = _sc_scatter(hs1l, hs1r, src2, dst2, zeros_nh)
  hs2l, hs2r = _tc_layer(p1, hs1l, hs1r, dinv, b1, g1, be1, W2)
  p2 = _sc_scatter(hs2l, hs2r, src2, dst2, zeros_nh)
  hs3l, hs3r = _tc_layer(p2, hs2l, hs2r, dinv, b2, g2, be2, W3)
  p3 = _sc_scatter(hs3l, hs3r, src2, dst2, zeros_nh)
  pa, pb = _tc_final(p3, hs3l, hs3r, dinv, b3, fc1_W, fc1_b)
  ga, gb, gn = _sc_gather3(pa, pb, src3, dst3, neg3)
  pos, negv = _tc_decode(ga, gb, gn, fc2_W, fc2_b.reshape(1, 1))
  return pos.reshape(-1), negv.reshape(-1)


# speedup vs baseline: 1.7582x; 1.0005x over previous
import functools

import jax
import jax.numpy as jnp
from jax import lax
from jax.experimental import pallas as pl
from jax.experimental.pallas import tpu as pltpu
from jax.experimental.pallas import tpu_sc as plsc

N = 10000
E = 320000
D = 128

NC = 2
NS = 16
NW = NC * NS
EPT = E // NW
WIN = 100
NWIN = EPT // WIN
H = D // 2
EPT2 = E // NS
NWIN2 = EPT2 // WIN

_mesh = plsc.VectorSubcoreMesh(core_axis_name="c", subcore_axis_name="s")
_untiled = pltpu.CompilerParams(use_tc_tiling_on_sc=False)


def _sc_degree(dst3, zeros_n16, ones_w16):
  @functools.partial(
      pl.kernel,
      mesh=_mesh,
      compiler_params=_untiled,
      out_type=jax.ShapeDtypeStruct((NC, N, 16), jnp.float32),
      scratch_types=[
          pltpu.VMEM((NWIN, WIN), jnp.int32),
          pltpu.VMEM((WIN, 16), jnp.float32),
          pltpu.VMEM_SHARED((N, 16), jnp.float32),
          pltpu.SemaphoreType.DMA,
      ],
  )
  def k(dst_hbm, zeros_hbm, ones_hbm, out_hbm, dst_v, ones_v, acc, sem):
    cid = lax.axis_index("c")
    sid = lax.axis_index("s")
    wid = sid * NC + cid

    @pl.when(sid == 0)
    def _():
      pltpu.sync_copy(zeros_hbm, acc)

    pltpu.sync_copy(dst_hbm.at[wid], dst_v)
    pltpu.sync_copy(ones_hbm, ones_v)
    plsc.subcore_barrier()

    @pl.loop(0, NWIN)
    def _(w):
      pltpu.async_copy(ones_v, acc.at[dst_v.at[w]], sem, add=True)

    @pl.loop(0, NWIN)
    def _(w):
      pltpu.make_async_copy(ones_v, acc.at[dst_v.at[0]], sem).wait()

    plsc.subcore_barrier()

    @pl.when(sid == 0)
    def _():
      pltpu.sync_copy(acc, out_hbm.at[cid])

  return k(dst3, zeros_n16, ones_w16)


def _sc_scatter(hsl, hsr, src2, dst2, zeros_nh):
  @functools.partial(
      pl.kernel,
      mesh=_mesh,
      compiler_params=_untiled,
      out_type=jax.ShapeDtypeStruct((NC, N, H), jnp.float32),
      scratch_types=[
          pltpu.VMEM((NWIN2, WIN), jnp.int32),
          pltpu.VMEM((NWIN2, WIN), jnp.int32),
          pltpu.VMEM((WIN, H), jnp.float32),
          pltpu.VMEM((WIN, H), jnp.float32),
          pltpu.VMEM((WIN, H), jnp.float32),
          pltpu.VMEM((WIN, H), jnp.float32),
          pltpu.VMEM_SHARED((N, H), jnp.float32),
          pltpu.SemaphoreType.DMA,
          pltpu.SemaphoreType.DMA,
          pltpu.SemaphoreType.DMA,
          pltpu.SemaphoreType.DMA,
          pltpu.SemaphoreType.DMA,
          pltpu.SemaphoreType.DMA,
          pltpu.SemaphoreType.DMA,
          pltpu.SemaphoreType.DMA,
      ],
  )
  def k(hsl_hbm, hsr_hbm, src_hbm, dst_hbm, zeros_hbm, out_hbm,
        src_v, dst_v, r0, r1, r2, r3, acc,
        g0, g1, g2, g3, s0, s1, s2, s3):
    cid = lax.axis_index("c")
    sid = lax.axis_index("s")
    rows = (r0, r1, r2, r3)
    gsem = (g0, g1, g2, g3)
    ssem = (s0, s1, s2, s3)

    @pl.when(sid == 0)
    def _():
      pltpu.sync_copy(zeros_hbm, acc)

    pltpu.sync_copy(src_hbm.at[sid], src_v)
    pltpu.sync_copy(dst_hbm.at[sid], dst_v)
    plsc.subcore_barrier()

    def run(table):
      for b in range(4):
        pltpu.async_copy(table.at[src_v.at[b]], rows[b], gsem[b])

      @pl.loop(0, NWIN2, step=4)
      def _(w):
        for b in range(4):
          pltpu.make_async_copy(
              table.at[src_v.at[w + b]], rows[b], gsem[b]).wait()
          pltpu.async_copy(
              rows[b], acc.at[dst_v.at[w + b]], ssem[b], add=True)
        for b in range(4):
          pltpu.make_async_copy(
              rows[b], acc.at[dst_v.at[w + b]], ssem[b]).wait()

          @pl.when(w + b + 4 < NWIN2)
          def _(b=b):
            pltpu.async_copy(
                table.at[src_v.at[w + b + 4]], rows[b], gsem[b])

    @pl.when(cid == 0)
    def _():
      run(hsl_hbm)

    @pl.when(cid == 1)
    def _():
      run(hsr_hbm)

    plsc.subcore_barrier()

    @pl.when(sid == 0)
    def _():
      pltpu.sync_copy(acc, out_hbm.at[cid])

  return k(hsl, hsr, src2, dst2, zeros_nh)


def _sc_gather3(pa, pb, src3, dst3, neg3, ne, nwin, win):
  @functools.partial(
      pl.kernel,
      mesh=_mesh,
      compiler_params=_untiled,
      out_type=(
          jax.ShapeDtypeStruct((ne, D), jnp.float32),
          jax.ShapeDtypeStruct((ne, D), jnp.float32),
          jax.ShapeDtypeStruct((ne, D), jnp.float32),
      ),
      scratch_types=[
          pltpu.VMEM((nwin, win), jnp.int32),
          pltpu.VMEM((win, D), jnp.float32),
          pltpu.VMEM((win, D), jnp.float32),
          pltpu.VMEM((win, D), jnp.float32),
          pltpu.VMEM((win, D), jnp.float32),
          pltpu.SemaphoreType.DMA,
          pltpu.SemaphoreType.DMA,
          pltpu.SemaphoreType.DMA,
          pltpu.SemaphoreType.DMA,
          pltpu.SemaphoreType.DMA,
          pltpu.SemaphoreType.DMA,
          pltpu.SemaphoreType.DMA,
          pltpu.SemaphoreType.DMA,
      ],
  )
  def k(pa_hbm, pb_hbm, src_hbm, dst_hbm, neg_hbm, ga_hbm, gb_hbm, gn_hbm,
        idx_v, r0, r1, r2, r3, g0, g1, g2, g3, s0, s1, s2, s3):
    cid = lax.axis_index("c")
    sid = lax.axis_index("s")
    wid = sid * NC + cid
    base = wid * (ne // NW)
    rows = (r0, r1, r2, r3)
    gsem = (g0, g1, g2, g3)
    ssem = (s0, s1, s2, s3)

    for table, idx_hbm, out_hbm in ((pa_hbm, src_hbm, ga_hbm),
                                    (pb_hbm, dst_hbm, gb_hbm),
                                    (pb_hbm, neg_hbm, gn_hbm)):
      pltpu.sync_copy(idx_hbm.at[wid], idx_v)
      for b in range(4):
        pltpu.async_copy(table.at[idx_v.at[b]], rows[b], gsem[b])

      @pl.loop(0, nwin, step=4)
      def _(w, table=table, out_hbm=out_hbm):
        for b in range(4):
          pltpu.make_async_copy(
              table.at[idx_v.at[w + b]], rows[b], gsem[b]).wait()
          pltpu.async_copy(
              rows[b], out_hbm.at[pl.ds(base + (w + b) * win, win)], ssem[b])
        for b in range(4):
          pltpu.make_async_copy(
              rows[b], out_hbm.at[pl.ds(base + (w + b) * win, win)],
              ssem[b]).wait()

          @pl.when(w + b + 4 < nwin)
          def _(b=b, table=table):
            pltpu.async_copy(table.at[idx_v.at[w + b + 4]], rows[b], gsem[b])

  return k(pa, pb, src3, dst3, neg3)


def _tc_prep_body(deg_ref, x_ref, w_ref, hsl_ref, hsr_ref, dinv_ref):
  deg = deg_ref[0] + deg_ref[1]
  total = deg[:, 0:1] + 1.0
  dinv = lax.rsqrt(total)
  h = jnp.dot(x_ref[...], w_ref[...], preferred_element_type=jnp.float32)
  hs = h * dinv
  hsl_ref[...] = hs[:, :H]
  hsr_ref[...] = hs[:, H:]
  dinv_ref[...] = dinv


def _tc_prep(deg_p, x, w1):
  return pl.pallas_call(
      _tc_prep_body,
      out_shape=(jax.ShapeDtypeStruct((N, H), jnp.float32),
                 jax.ShapeDtypeStruct((N, H), jnp.float32),
                 jax.ShapeDtypeStruct((N, 1), jnp.float32)),
  )(deg_p, x, w1)


def _node_state(p_ref, hsl_ref, hsr_ref, dinv, b_ref):
  hs = jnp.concatenate([hsl_ref[...], hsr_ref[...]], axis=1)
  psum = jnp.concatenate([p_ref[0], p_ref[1]], axis=1)
  return dinv * (psum + hs) + b_ref[...]


def _tc_layer_body(p_ref, hsl_ref, hsr_ref, dinv_ref, b_ref, g_ref, be_ref,
                   wn_ref, outl_ref, outr_ref):
  dinv = dinv_ref[...]
  t = _node_state(p_ref, hsl_ref, hsr_ref, dinv, b_ref)
  m = jnp.mean(t, axis=0, keepdims=True)
  c = t - m
  v = jnp.mean(c * c, axis=0, keepdims=True)
  y = c * lax.rsqrt(v + 1e-5) * g_ref[...] + be_ref[...]
  y = jnp.maximum(y, 0.0)
  hs = jnp.dot(y, wn_ref[...], preferred_element_type=jnp.float32) * dinv
  outl_ref[...] = hs[:, :H]
  outr_ref[...] = hs[:, H:]


def _tc_layer(p, hsl, hsr, dinv, b, g, be, wn):
  return pl.pallas_call(
      _tc_layer_body,
      out_shape=(jax.ShapeDtypeStruct((N, H), jnp.float32),
                 jax.ShapeDtypeStruct((N, H), jnp.float32)),
  )(p, hsl, hsr, dinv, b, g, be, wn)


def _tc_final_body(p_ref, hsl_ref, hsr_ref, dinv_ref, b_ref, w1_ref, b1_ref,
                   pa_ref, pb_ref):
  dinv = dinv_ref[...]
  z = _node_state(p_ref, hsl_ref, hsr_ref, dinv, b_ref)
  pa_ref[...] = jnp.dot(z, w1_ref[:D],
                        preferred_element_type=jnp.float32) + b1_ref[...]
  pb_ref[...] = jnp.dot(z, w1_ref[D:], preferred_element_type=jnp.float32)


def _tc_final(p, hsl, hsr, dinv, b3, fc1_w, fc1_b):
  return pl.pallas_call(
      _tc_final_body,
      out_shape=(jax.ShapeDtypeStruct((N, D), jnp.float32),
                 jax.ShapeDtypeStruct((N, D), jnp.float32)),
  )(p, hsl, hsr, dinv, b3, fc1_w, fc1_b)


_BE = 8000


def _tc_decode_body(ga_ref, gb_ref, gn_ref, w_ref, c_ref, pos_ref, neg_ref):
  ga = ga_ref[...]
  w = w_ref[...]
  c = c_ref[0, 0]
  hp = jnp.maximum(ga + gb_ref[...], 0.0)
  hn = jnp.maximum(ga + gn_ref[...], 0.0)
  pos_ref[...] = jax.nn.sigmoid(
      jnp.dot(hp, w, preferred_element_type=jnp.float32) + c)
  neg_ref[...] = jax.nn.sigmoid(
      jnp.dot(hn, w, preferred_element_type=jnp.float32) + c)


def _tc_decode(ga, gb, gn, fc2_w, fc2_b):
  ne = ga.shape[0]
  grid = (ne // _BE,)
  edge_spec = pl.BlockSpec((_BE, D), lambda i: (i, 0))
  return pl.pallas_call(
      _tc_decode_body,
      grid=grid,
      in_specs=[edge_spec, edge_spec, edge_spec,
                pl.BlockSpec((D, 1), lambda i: (0, 0)),
                pl.BlockSpec((1, 1), lambda i: (0, 0))],
      out_specs=(pl.BlockSpec((_BE, 1), lambda i: (i, 0)),
                 pl.BlockSpec((_BE, 1), lambda i: (i, 0))),
      out_shape=(jax.ShapeDtypeStruct((ne, 1), jnp.float32),
                 jax.ShapeDtypeStruct((ne, 1), jnp.float32)),
  )(ga, gb, gn, fc2_w, fc2_b)


def kernel(node_feat, src, dst, neg, W1, b1, W2, b2, W3, b3,
           g1, be1, g2, be2, fc1_W, fc1_b, fc2_W, fc2_b):
  src3 = src.reshape(NW, NWIN, WIN)
  dst3 = dst.reshape(NW, NWIN, WIN)
  neg3 = neg.reshape(NW, NWIN, WIN)
  src2 = src.reshape(NS, NWIN2, WIN)
  dst2 = dst.reshape(NS, NWIN2, WIN)

  zeros_nh = jnp.zeros((N, H), jnp.float32)
  zeros_n16 = jnp.zeros((N, 16), jnp.float32)
  ones_w16 = jnp.ones((WIN, 16), jnp.float32)

  deg_p = _sc_degree(dst3, zeros_n16, ones_w16)
  hs1l, hs1r, dinv = _tc_prep(deg_p, node_feat, W1)
  p1 = _sc_scatter(hs1l, hs1r, src2, dst2, zeros_nh)
  hs2l, hs2r = _tc_layer(p1, hs1l, hs1r, dinv, b1, g1, be1, W2)
  p2 = _sc_scatter(hs2l, hs2r, src2, dst2, zeros_nh)
  hs3l, hs3r = _tc_layer(p2, hs2l, hs2r, dinv, b2, g2, be2, W3)
  p3 = _sc_scatter(hs3l, hs3r, src2, dst2, zeros_nh)
  pa, pb = _tc_final(p3, hs3l, hs3r, dinv, b3, fc1_W, fc1_b)
  e2 = E // 2
  winh = 125
  nwinh = e2 // NW // winh
  srch = src.reshape(2, NW, nwinh, winh)
  dsth = dst.reshape(2, NW, nwinh, winh)
  negh = neg.reshape(2, NW, nwinh, winh)
  fb = fc2_b.reshape(1, 1)
  ga0, gb0, gn0 = _sc_gather3(pa, pb, srch[0], dsth[0], negh[0], e2, nwinh, winh)
  ga1, gb1, gn1 = _sc_gather3(pa, pb, srch[1], dsth[1], negh[1], e2, nwinh, winh)
  pos0, neg0 = _tc_decode(ga0, gb0, gn0, fc2_W, fb)
  pos1, neg1 = _tc_decode(ga1, gb1, gn1, fc2_W, fb)
  pos = jnp.concatenate([pos0.reshape(-1), pos1.reshape(-1)])
  negv = jnp.concatenate([neg0.reshape(-1), neg1.reshape(-1)])
  return pos, negv


# speedup vs baseline: 1.9407x; 1.1038x over previous
import functools

import jax
import jax.numpy as jnp
from jax import lax
from jax.experimental import pallas as pl
from jax.experimental.pallas import tpu as pltpu
from jax.experimental.pallas import tpu_sc as plsc

N = 10000
E = 320000
D = 128

NC = 2
NS = 16
NW = NC * NS
EPT = E // NW
WIN = 100
NWIN = EPT // WIN
H = D // 2
EPT2 = E // NS
NWIN2 = EPT2 // WIN

_mesh = plsc.VectorSubcoreMesh(core_axis_name="c", subcore_axis_name="s")
_untiled = pltpu.CompilerParams(use_tc_tiling_on_sc=False)


def _sc_degree(dst3, zeros_n16, ones_w16):
  @functools.partial(
      pl.kernel,
      mesh=_mesh,
      compiler_params=_untiled,
      out_type=jax.ShapeDtypeStruct((NC, N, 16), jnp.float32),
      scratch_types=[
          pltpu.VMEM((NWIN, WIN), jnp.int32),
          pltpu.VMEM((WIN, 16), jnp.float32),
          pltpu.VMEM_SHARED((N, 16), jnp.float32),
          pltpu.SemaphoreType.DMA,
      ],
  )
  def k(dst_hbm, zeros_hbm, ones_hbm, out_hbm, dst_v, ones_v, acc, sem):
    cid = lax.axis_index("c")
    sid = lax.axis_index("s")
    wid = sid * NC + cid

    @pl.when(sid == 0)
    def _():
      pltpu.sync_copy(zeros_hbm, acc)

    pltpu.sync_copy(dst_hbm.at[wid], dst_v)
    pltpu.sync_copy(ones_hbm, ones_v)
    plsc.subcore_barrier()

    @pl.loop(0, NWIN)
    def _(w):
      pltpu.async_copy(ones_v, acc.at[dst_v.at[w]], sem, add=True)

    @pl.loop(0, NWIN)
    def _(w):
      pltpu.make_async_copy(ones_v, acc.at[dst_v.at[0]], sem).wait()

    plsc.subcore_barrier()

    @pl.when(sid == 0)
    def _():
      pltpu.sync_copy(acc, out_hbm.at[cid])

  return k(dst3, zeros_n16, ones_w16)


def _sc_scatter(hsl, hsr, src2, dst2, zeros_nh):
  @functools.partial(
      pl.kernel,
      mesh=_mesh,
      compiler_params=_untiled,
      out_type=jax.ShapeDtypeStruct((NC, N, H), jnp.float32),
      scratch_types=[
          pltpu.VMEM((NWIN2, WIN), jnp.int32),
          pltpu.VMEM((NWIN2, WIN), jnp.int32),
          pltpu.VMEM((WIN, H), jnp.float32),
          pltpu.VMEM((WIN, H), jnp.float32),
          pltpu.VMEM((WIN, H), jnp.float32),
          pltpu.VMEM((WIN, H), jnp.float32),
          pltpu.VMEM_SHARED((N, H), jnp.float32),
          pltpu.SemaphoreType.DMA,
          pltpu.SemaphoreType.DMA,
          pltpu.SemaphoreType.DMA,
          pltpu.SemaphoreType.DMA,
          pltpu.SemaphoreType.DMA,
          pltpu.SemaphoreType.DMA,
          pltpu.SemaphoreType.DMA,
          pltpu.SemaphoreType.DMA,
      ],
  )
  def k(hsl_hbm, hsr_hbm, src_hbm, dst_hbm, zeros_hbm, out_hbm,
        src_v, dst_v, r0, r1, r2, r3, acc,
        g0, g1, g2, g3, s0, s1, s2, s3):
    cid = lax.axis_index("c")
    sid = lax.axis_index("s")
    rows = (r0, r1, r2, r3)
    gsem = (g0, g1, g2, g3)
    ssem = (s0, s1, s2, s3)

    @pl.when(sid == 0)
    def _():
      pltpu.sync_copy(zeros_hbm, acc)

    pltpu.sync_copy(src_hbm.at[sid], src_v)
    pltpu.sync_copy(dst_hbm.at[sid], dst_v)
    plsc.subcore_barrier()

    def run(table):
      for b in range(4):
        pltpu.async_copy(table.at[src_v.at[b]], rows[b], gsem[b])

      @pl.loop(0, NWIN2, step=4)
      def _(w):
        for b in range(4):
          pltpu.make_async_copy(
              table.at[src_v.at[w + b]], rows[b], gsem[b]).wait()
          pltpu.async_copy(
              rows[b], acc.at[dst_v.at[w + b]], ssem[b], add=True)
        for b in range(4):
          pltpu.make_async_copy(
              rows[b], acc.at[dst_v.at[w + b]], ssem[b]).wait()

          @pl.when(w + b + 4 < NWIN2)
          def _(b=b):
            pltpu.async_copy(
                table.at[src_v.at[w + b + 4]], rows[b], gsem[b])

    @pl.when(cid == 0)
    def _():
      run(hsl_hbm)

    @pl.when(cid == 1)
    def _():
      run(hsr_hbm)

    plsc.subcore_barrier()

    @pl.when(sid == 0)
    def _():
      pltpu.sync_copy(acc, out_hbm.at[cid])

  return k(hsl, hsr, src2, dst2, zeros_nh)


def _sc_gather3(pa, pb, src3, dst3, neg3, ne, nwin, win):
  @functools.partial(
      pl.kernel,
      mesh=_mesh,
      compiler_params=_untiled,
      out_type=(
          jax.ShapeDtypeStruct((ne, D), jnp.float32),
          jax.ShapeDtypeStruct((ne, D), jnp.float32),
          jax.ShapeDtypeStruct((ne, D), jnp.float32),
      ),
      scratch_types=[
          pltpu.VMEM((nwin, win), jnp.int32),
          pltpu.VMEM((win, D), jnp.float32),
          pltpu.VMEM((win, D), jnp.float32),
          pltpu.VMEM((win, D), jnp.float32),
          pltpu.VMEM((win, D), jnp.float32),
          pltpu.SemaphoreType.DMA,
          pltpu.SemaphoreType.DMA,
          pltpu.SemaphoreType.DMA,
          pltpu.SemaphoreType.DMA,
          pltpu.SemaphoreType.DMA,
          pltpu.SemaphoreType.DMA,
          pltpu.SemaphoreType.DMA,
          pltpu.SemaphoreType.DMA,
      ],
  )
  def k(pa_hbm, pb_hbm, src_hbm, dst_hbm, neg_hbm, ga_hbm, gb_hbm, gn_hbm,
        idx_v, r0, r1, r2, r3, g0, g1, g2, g3, s0, s1, s2, s3):
    cid = lax.axis_index("c")
    sid = lax.axis_index("s")
    wid = sid * NC + cid
    base = wid * (ne // NW)
    rows = (r0, r1, r2, r3)
    gsem = (g0, g1, g2, g3)
    ssem = (s0, s1, s2, s3)

    for table, idx_hbm, out_hbm in ((pa_hbm, src_hbm, ga_hbm),
                                    (pb_hbm, dst_hbm, gb_hbm),
                                    (pb_hbm, neg_hbm, gn_hbm)):
      pltpu.sync_copy(idx_hbm.at[wid], idx_v)
      for b in range(4):
        pltpu.async_copy(table.at[idx_v.at[b]], rows[b], gsem[b])

      @pl.loop(0, nwin, step=4)
      def _(w, table=table, out_hbm=out_hbm):
        for b in range(4):
          pltpu.make_async_copy(
              table.at[idx_v.at[w + b]], rows[b], gsem[b]).wait()
          pltpu.async_copy(
              rows[b], out_hbm.at[pl.ds(base + (w + b) * win, win)], ssem[b])
        for b in range(4):
          pltpu.make_async_copy(
              rows[b], out_hbm.at[pl.ds(base + (w + b) * win, win)],
              ssem[b]).wait()

          @pl.when(w + b + 4 < nwin)
          def _(b=b, table=table):
            pltpu.async_copy(table.at[idx_v.at[w + b + 4]], rows[b], gsem[b])

  return k(pa, pb, src3, dst3, neg3)


def _tc_prep_body(deg_ref, x_ref, w_ref, hsl_ref, hsr_ref, dinv_ref):
  deg = deg_ref[0] + deg_ref[1]
  total = deg[:, 0:1] + 1.0
  dinv = lax.rsqrt(total)
  h = jnp.dot(x_ref[...], w_ref[...], preferred_element_type=jnp.float32)
  hs = h * dinv
  hsl_ref[...] = hs[:, :H]
  hsr_ref[...] = hs[:, H:]
  dinv_ref[...] = dinv


def _tc_prep(deg_p, x, w1):
  return pl.pallas_call(
      _tc_prep_body,
      out_shape=(jax.ShapeDtypeStruct((N, H), jnp.float32),
                 jax.ShapeDtypeStruct((N, H), jnp.float32),
                 jax.ShapeDtypeStruct((N, 1), jnp.float32)),
  )(deg_p, x, w1)


def _node_state(p_ref, hsl_ref, hsr_ref, dinv, b_ref):
  hs = jnp.concatenate([hsl_ref[...], hsr_ref[...]], axis=1)
  psum = jnp.concatenate([p_ref[0], p_ref[1]], axis=1)
  return dinv * (psum + hs) + b_ref[...]


def _tc_layer_body(p_ref, hsl_ref, hsr_ref, dinv_ref, b_ref, g_ref, be_ref,
                   wn_ref, outl_ref, outr_ref):
  dinv = dinv_ref[...]
  t = _node_state(p_ref, hsl_ref, hsr_ref, dinv, b_ref)
  m = jnp.mean(t, axis=0, keepdims=True)
  c = t - m
  v = jnp.mean(c * c, axis=0, keepdims=True)
  y = c * lax.rsqrt(v + 1e-5) * g_ref[...] + be_ref[...]
  y = jnp.maximum(y, 0.0)
  hs = jnp.dot(y, wn_ref[...], preferred_element_type=jnp.float32) * dinv
  outl_ref[...] = hs[:, :H]
  outr_ref[...] = hs[:, H:]


def _tc_layer(p, hsl, hsr, dinv, b, g, be, wn):
  return pl.pallas_call(
      _tc_layer_body,
      out_shape=(jax.ShapeDtypeStruct((N, H), jnp.float32),
                 jax.ShapeDtypeStruct((N, H), jnp.float32)),
  )(p, hsl, hsr, dinv, b, g, be, wn)


def _tc_final_body(p_ref, hsl_ref, hsr_ref, dinv_ref, b_ref, w1_ref, b1_ref,
                   pa_ref, pb_ref):
  dinv = dinv_ref[...]
  z = _node_state(p_ref, hsl_ref, hsr_ref, dinv, b_ref)
  pa_ref[...] = jnp.dot(z, w1_ref[:D],
                        preferred_element_type=jnp.float32) + b1_ref[...]
  pb_ref[...] = jnp.dot(z, w1_ref[D:], preferred_element_type=jnp.float32)


def _tc_final(p, hsl, hsr, dinv, b3, fc1_w, fc1_b):
  return pl.pallas_call(
      _tc_final_body,
      out_shape=(jax.ShapeDtypeStruct((N, D), jnp.float32),
                 jax.ShapeDtypeStruct((N, D), jnp.float32)),
  )(p, hsl, hsr, dinv, b3, fc1_w, fc1_b)


_BE = 6400


def _tc_decode_body(ga_ref, gb_ref, gn_ref, w_ref, c_ref, pos_ref, neg_ref):
  i = pl.program_id(0)
  ga = ga_ref[...]
  w = w_ref[...]
  c = c_ref[0, 0]
  hp = jnp.maximum(ga + gb_ref[...], 0.0)
  hn = jnp.maximum(ga + gn_ref[...], 0.0)
  pos_ref[pl.ds(i * _BE, _BE)] = jax.nn.sigmoid(jnp.sum(hp * w, axis=1) + c)
  neg_ref[pl.ds(i * _BE, _BE)] = jax.nn.sigmoid(jnp.sum(hn * w, axis=1) + c)


def _tc_decode(ga, gb, gn, fc2_w, fc2_b):
  ne = ga.shape[0]
  grid = (ne // _BE,)
  edge_spec = pl.BlockSpec((_BE, D), lambda i: (i, 0))
  return pl.pallas_call(
      _tc_decode_body,
      grid=grid,
      in_specs=[edge_spec, edge_spec, edge_spec,
                pl.BlockSpec((1, D), lambda i: (0, 0)),
                pl.BlockSpec((1, 1), lambda i: (0, 0))],
      out_specs=(pl.BlockSpec((ne,), lambda i: (0,)),
                 pl.BlockSpec((ne,), lambda i: (0,))),
      out_shape=(jax.ShapeDtypeStruct((ne,), jnp.float32),
                 jax.ShapeDtypeStruct((ne,), jnp.float32)),
  )(ga, gb, gn, fc2_w, fc2_b)


def kernel(node_feat, src, dst, neg, W1, b1, W2, b2, W3, b3,
           g1, be1, g2, be2, fc1_W, fc1_b, fc2_W, fc2_b):
  src3 = src.reshape(NW, NWIN, WIN)
  dst3 = dst.reshape(NW, NWIN, WIN)
  neg3 = neg.reshape(NW, NWIN, WIN)
  src2 = src.reshape(NS, NWIN2, WIN)
  dst2 = dst.reshape(NS, NWIN2, WIN)

  zeros_nh = jnp.zeros((N, H), jnp.float32)
  zeros_n16 = jnp.zeros((N, 16), jnp.float32)
  ones_w16 = jnp.ones((WIN, 16), jnp.float32)

  deg_p = _sc_degree(dst3, zeros_n16, ones_w16)
  hs1l, hs1r, dinv = _tc_prep(deg_p, node_feat, W1)
  p1 = _sc_scatter(hs1l, hs1r, src2, dst2, zeros_nh)
  hs2l, hs2r = _tc_layer(p1, hs1l, hs1r, dinv, b1, g1, be1, W2)
  p2 = _sc_scatter(hs2l, hs2r, src2, dst2, zeros_nh)
  hs3l, hs3r = _tc_layer(p2, hs2l, hs2r, dinv, b2, g2, be2, W3)
  p3 = _sc_scatter(hs3l, hs3r, src2, dst2, zeros_nh)
  pa, pb = _tc_final(p3, hs3l, hs3r, dinv, b3, fc1_W, fc1_b)
  e2 = E // 2
  winh = 125
  nwinh = e2 // NW // winh
  srch = src.reshape(2, NW, nwinh, winh)
  dsth = dst.reshape(2, NW, nwinh, winh)
  negh = neg.reshape(2, NW, nwinh, winh)
  fb = fc2_b.reshape(1, 1)
  fw = fc2_W.reshape(1, D)
  ga0, gb0, gn0 = _sc_gather3(pa, pb, srch[0], dsth[0], negh[0], e2, nwinh, winh)
  ga1, gb1, gn1 = _sc_gather3(pa, pb, srch[1], dsth[1], negh[1], e2, nwinh, winh)
  pos0, neg0 = _tc_decode(ga0, gb0, gn0, fw, fb)
  pos1, neg1 = _tc_decode(ga1, gb1, gn1, fw, fb)
  pos = jnp.concatenate([pos0, pos1])
  negv = jnp.concatenate([neg0, neg1])
  return pos, negv


# speedup vs baseline: 1.9424x; 1.0008x over previous
import functools

import jax
import jax.numpy as jnp
from jax import lax
from jax.experimental import pallas as pl
from jax.experimental.pallas import tpu as pltpu
from jax.experimental.pallas import tpu_sc as plsc

N = 10000
E = 320000
D = 128

NC = 2
NS = 16
NW = NC * NS
EPT = E // NW
WIN = 100
NWIN = EPT // WIN
H = D // 2
EPT2 = E // NS
WIN2 = 125
NWIN2 = EPT2 // WIN2

_mesh = plsc.VectorSubcoreMesh(core_axis_name="c", subcore_axis_name="s")
_untiled = pltpu.CompilerParams(use_tc_tiling_on_sc=False)


def _sc_degree(dst3, zeros_n16, ones_w16):
  @functools.partial(
      pl.kernel,
      mesh=_mesh,
      compiler_params=_untiled,
      out_type=jax.ShapeDtypeStruct((NC, N, 16), jnp.float32),
      scratch_types=[
          pltpu.VMEM((NWIN, WIN), jnp.int32),
          pltpu.VMEM((WIN, 16), jnp.float32),
          pltpu.VMEM_SHARED((N, 16), jnp.float32),
          pltpu.SemaphoreType.DMA,
      ],
  )
  def k(dst_hbm, zeros_hbm, ones_hbm, out_hbm, dst_v, ones_v, acc, sem):
    cid = lax.axis_index("c")
    sid = lax.axis_index("s")
    wid = sid * NC + cid

    @pl.when(sid == 0)
    def _():
      pltpu.sync_copy(zeros_hbm, acc)

    pltpu.sync_copy(dst_hbm.at[wid], dst_v)
    pltpu.sync_copy(ones_hbm, ones_v)
    plsc.subcore_barrier()

    @pl.loop(0, NWIN)
    def _(w):
      pltpu.async_copy(ones_v, acc.at[dst_v.at[w]], sem, add=True)

    @pl.loop(0, NWIN)
    def _(w):
      pltpu.make_async_copy(ones_v, acc.at[dst_v.at[0]], sem).wait()

    plsc.subcore_barrier()

    @pl.when(sid == 0)
    def _():
      pltpu.sync_copy(acc, out_hbm.at[cid])

  return k(dst3, zeros_n16, ones_w16)


def _sc_scatter(hsl, hsr, src2, dst2, zeros_nh):
  @functools.partial(
      pl.kernel,
      mesh=_mesh,
      compiler_params=_untiled,
      out_type=jax.ShapeDtypeStruct((NC, N, H), jnp.float32),
      scratch_types=[
          pltpu.VMEM((NWIN2, WIN2), jnp.int32),
          pltpu.VMEM((NWIN2, WIN2), jnp.int32),
          pltpu.VMEM((WIN2, H), jnp.float32),
          pltpu.VMEM((WIN2, H), jnp.float32),
          pltpu.VMEM((WIN2, H), jnp.float32),
          pltpu.VMEM((WIN2, H), jnp.float32),
          pltpu.VMEM_SHARED((N, H), jnp.float32),
          pltpu.SemaphoreType.DMA,
          pltpu.SemaphoreType.DMA,
          pltpu.SemaphoreType.DMA,
          pltpu.SemaphoreType.DMA,
          pltpu.SemaphoreType.DMA,
          pltpu.SemaphoreType.DMA,
          pltpu.SemaphoreType.DMA,
          pltpu.SemaphoreType.DMA,
      ],
  )
  def k(hsl_hbm, hsr_hbm, src_hbm, dst_hbm, zeros_hbm, out_hbm,
        src_v, dst_v, r0, r1, r2, r3, acc,
        g0, g1, g2, g3, s0, s1, s2, s3):
    cid = lax.axis_index("c")
    sid = lax.axis_index("s")
    rows = (r0, r1, r2, r3)
    gsem = (g0, g1, g2, g3)
    ssem = (s0, s1, s2, s3)

    @pl.when(sid == 0)
    def _():
      pltpu.sync_copy(zeros_hbm, acc)

    pltpu.sync_copy(src_hbm.at[sid], src_v)
    pltpu.sync_copy(dst_hbm.at[sid], dst_v)
    plsc.subcore_barrier()

    def run(table):
      for b in range(4):
        pltpu.async_copy(table.at[src_v.at[b]], rows[b], gsem[b])

      @pl.loop(0, NWIN2, step=4)
      def _(w):
        for b in range(4):
          pltpu.make_async_copy(
              table.at[src_v.at[w + b]], rows[b], gsem[b]).wait()
          pltpu.async_copy(
              rows[b], acc.at[dst_v.at[w + b]], ssem[b], add=True)
        for b in range(4):
          pltpu.make_async_copy(
              rows[b], acc.at[dst_v.at[w + b]], ssem[b]).wait()

          @pl.when(w + b + 4 < NWIN2)
          def _(b=b):
            pltpu.async_copy(
                table.at[src_v.at[w + b + 4]], rows[b], gsem[b])

    @pl.when(cid == 0)
    def _():
      run(hsl_hbm)

    @pl.when(cid == 1)
    def _():
      run(hsr_hbm)

    plsc.subcore_barrier()

    @pl.when(sid == 0)
    def _():
      pltpu.sync_copy(acc, out_hbm.at[cid])

  return k(hsl, hsr, src2, dst2, zeros_nh)


def _sc_gather3(pa, pb, src3, dst3, neg3, ne, nwin, win):
  @functools.partial(
      pl.kernel,
      mesh=_mesh,
      compiler_params=_untiled,
      out_type=(
          jax.ShapeDtypeStruct((ne, D), jnp.float32),
          jax.ShapeDtypeStruct((ne, D), jnp.float32),
          jax.ShapeDtypeStruct((ne, D), jnp.float32),
      ),
      scratch_types=[
          pltpu.VMEM((nwin, win), jnp.int32),
          pltpu.VMEM((win, D), jnp.float32),
          pltpu.VMEM((win, D), jnp.float32),
          pltpu.VMEM((win, D), jnp.float32),
          pltpu.VMEM((win, D), jnp.float32),
          pltpu.SemaphoreType.DMA,
          pltpu.SemaphoreType.DMA,
          pltpu.SemaphoreType.DMA,
          pltpu.SemaphoreType.DMA,
          pltpu.SemaphoreType.DMA,
          pltpu.SemaphoreType.DMA,
          pltpu.SemaphoreType.DMA,
          pltpu.SemaphoreType.DMA,
      ],
  )
  def k(pa_hbm, pb_hbm, src_hbm, dst_hbm, neg_hbm, ga_hbm, gb_hbm, gn_hbm,
        idx_v, r0, r1, r2, r3, g0, g1, g2, g3, s0, s1, s2, s3):
    cid = lax.axis_index("c")
    sid = lax.axis_index("s")
    wid = sid * NC + cid
    base = wid * (ne // NW)
    rows = (r0, r1, r2, r3)
    gsem = (g0, g1, g2, g3)
    ssem = (s0, s1, s2, s3)

    for table, idx_hbm, out_hbm in ((pa_hbm, src_hbm, ga_hbm),
                                    (pb_hbm, dst_hbm, gb_hbm),
                                    (pb_hbm, neg_hbm, gn_hbm)):
      pltpu.sync_copy(idx_hbm.at[wid], idx_v)
      for b in range(4):
        pltpu.async_copy(table.at[idx_v.at[b]], rows[b], gsem[b])

      @pl.loop(0, nwin, step=4)
      def _(w, table=table, out_hbm=out_hbm):
        for b in range(4):
          pltpu.make_async_copy(
              table.at[idx_v.at[w + b]], rows[b], gsem[b]).wait()
          pltpu.async_copy(
              rows[b], out_hbm.at[pl.ds(base + (w + b) * win, win)], ssem[b])
        for b in range(4):
          pltpu.make_async_copy(
              rows[b], out_hbm.at[pl.ds(base + (w + b) * win, win)],
              ssem[b]).wait()

          @pl.when(w + b + 4 < nwin)
          def _(b=b, table=table):
            pltpu.async_copy(table.at[idx_v.at[w + b + 4]], rows[b], gsem[b])

  return k(pa, pb, src3, dst3, neg3)


def _tc_prep_body(deg_ref, x_ref, w_ref, hsl_ref, hsr_ref, dinv_ref):
  deg = deg_ref[0] + deg_ref[1]
  total = deg[:, 0:1] + 1.0
  dinv = lax.rsqrt(total)
  h = jnp.dot(x_ref[...], w_ref[...], preferred_element_type=jnp.float32)
  hs = h * dinv
  hsl_ref[...] = hs[:, :H]
  hsr_ref[...] = hs[:, H:]
  dinv_ref[...] = dinv


def _tc_prep(deg_p, x, w1):
  return pl.pallas_call(
      _tc_prep_body,
      out_shape=(jax.ShapeDtypeStruct((N, H), jnp.float32),
                 jax.ShapeDtypeStruct((N, H), jnp.float32),
                 jax.ShapeDtypeStruct((N, 1), jnp.float32)),
  )(deg_p, x, w1)


def _node_state(p_ref, hsl_ref, hsr_ref, dinv, b_ref):
  hs = jnp.concatenate([hsl_ref[...], hsr_ref[...]], axis=1)
  psum = jnp.concatenate([p_ref[0], p_ref[1]], axis=1)
  return dinv * (psum + hs) + b_ref[...]


def _tc_layer_body(p_ref, hsl_ref, hsr_ref, dinv_ref, b_ref, g_ref, be_ref,
                   wn_ref, outl_ref, outr_ref):
  dinv = dinv_ref[...]
  t = _node_state(p_ref, hsl_ref, hsr_ref, dinv, b_ref)
  m = jnp.mean(t, axis=0, keepdims=True)
  c = t - m
  v = jnp.mean(c * c, axis=0, keepdims=True)
  y = c * lax.rsqrt(v + 1e-5) * g_ref[...] + be_ref[...]
  y = jnp.maximum(y, 0.0)
  hs = jnp.dot(y, wn_ref[...], preferred_element_type=jnp.float32) * dinv
  outl_ref[...] = hs[:, :H]
  outr_ref[...] = hs[:, H:]


def _tc_layer(p, hsl, hsr, dinv, b, g, be, wn):
  return pl.pallas_call(
      _tc_layer_body,
      out_shape=(jax.ShapeDtypeStruct((N, H), jnp.float32),
                 jax.ShapeDtypeStruct((N, H), jnp.float32)),
  )(p, hsl, hsr, dinv, b, g, be, wn)


def _tc_final_body(p_ref, hsl_ref, hsr_ref, dinv_ref, b_ref, w1_ref, b1_ref,
                   pa_ref, pb_ref):
  dinv = dinv_ref[...]
  z = _node_state(p_ref, hsl_ref, hsr_ref, dinv, b_ref)
  pa_ref[...] = jnp.dot(z, w1_ref[:D],
                        preferred_element_type=jnp.float32) + b1_ref[...]
  pb_ref[...] = jnp.dot(z, w1_ref[D:], preferred_element_type=jnp.float32)


def _tc_final(p, hsl, hsr, dinv, b3, fc1_w, fc1_b):
  return pl.pallas_call(
      _tc_final_body,
      out_shape=(jax.ShapeDtypeStruct((N, D), jnp.float32),
                 jax.ShapeDtypeStruct((N, D), jnp.float32)),
  )(p, hsl, hsr, dinv, b3, fc1_w, fc1_b)


_BE = 6400


def _tc_decode_body(ga_ref, gb_ref, gn_ref, w_ref, c_ref, pos_ref, neg_ref):
  i = pl.program_id(0)
  ga = ga_ref[...]
  w = w_ref[...]
  c = c_ref[0, 0]
  hp = jnp.maximum(ga + gb_ref[...], 0.0)
  hn = jnp.maximum(ga + gn_ref[...], 0.0)
  pos_ref[pl.ds(i * _BE, _BE)] = jax.nn.sigmoid(jnp.sum(hp * w, axis=1) + c)
  neg_ref[pl.ds(i * _BE, _BE)] = jax.nn.sigmoid(jnp.sum(hn * w, axis=1) + c)


def _tc_decode(ga, gb, gn, fc2_w, fc2_b):
  ne = ga.shape[0]
  grid = (ne // _BE,)
  edge_spec = pl.BlockSpec((_BE, D), lambda i: (i, 0))
  return pl.pallas_call(
      _tc_decode_body,
      grid=grid,
      in_specs=[edge_spec, edge_spec, edge_spec,
                pl.BlockSpec((1, D), lambda i: (0, 0)),
                pl.BlockSpec((1, 1), lambda i: (0, 0))],
      out_specs=(pl.BlockSpec((ne,), lambda i: (0,)),
                 pl.BlockSpec((ne,), lambda i: (0,))),
      out_shape=(jax.ShapeDtypeStruct((ne,), jnp.float32),
                 jax.ShapeDtypeStruct((ne,), jnp.float32)),
  )(ga, gb, gn, fc2_w, fc2_b)


def kernel(node_feat, src, dst, neg, W1, b1, W2, b2, W3, b3,
           g1, be1, g2, be2, fc1_W, fc1_b, fc2_W, fc2_b):
  src3 = src.reshape(NW, NWIN, WIN)
  dst3 = dst.reshape(NW, NWIN, WIN)
  neg3 = neg.reshape(NW, NWIN, WIN)
  src2 = src.reshape(NS, NWIN2, WIN2)
  dst2 = dst.reshape(NS, NWIN2, WIN2)

  zeros_nh = jnp.zeros((N, H), jnp.float32)
  zeros_n16 = jnp.zeros((N, 16), jnp.float32)
  ones_w16 = jnp.ones((WIN, 16), jnp.float32)

  deg_p = _sc_degree(dst3, zeros_n16, ones_w16)
  hs1l, hs1r, dinv = _tc_prep(deg_p, node_feat, W1)
  p1 = _sc_scatter(hs1l, hs1r, src2, dst2, zeros_nh)
  hs2l, hs2r = _tc_layer(p1, hs1l, hs1r, dinv, b1, g1, be1, W2)
  p2 = _sc_scatter(hs2l, hs2r, src2, dst2, zeros_nh)
  hs3l, hs3r = _tc_layer(p2, hs2l, hs2r, dinv, b2, g2, be2, W3)
  p3 = _sc_scatter(hs3l, hs3r, src2, dst2, zeros_nh)
  pa, pb = _tc_final(p3, hs3l, hs3r, dinv, b3, fc1_W, fc1_b)
  e2 = E // 2
  winh = 125
  nwinh = e2 // NW // winh
  srch = src.reshape(2, NW, nwinh, winh)
  dsth = dst.reshape(2, NW, nwinh, winh)
  negh = neg.reshape(2, NW, nwinh, winh)
  fb = fc2_b.reshape(1, 1)
  fw = fc2_W.reshape(1, D)
  ga0, gb0, gn0 = _sc_gather3(pa, pb, srch[0], dsth[0], negh[0], e2, nwinh, winh)
  ga1, gb1, gn1 = _sc_gather3(pa, pb, srch[1], dsth[1], negh[1], e2, nwinh, winh)
  pos0, neg0 = _tc_decode(ga0, gb0, gn0, fw, fb)
  pos1, neg1 = _tc_decode(ga1, gb1, gn1, fw, fb)
  pos = jnp.concatenate([pos0, pos1])
  negv = jnp.concatenate([neg0, neg1])
  return pos, negv
